# Initial kernel scaffold; baseline (speedup 1.0000x reference)
import functools

import jax
import jax.numpy as jnp
from jax import lax
from jax.experimental import pallas as pl
from jax.experimental.pallas import tpu as pltpu
from jax.experimental.pallas import tpu_sc as plsc

N_NODES = 10000
N_EDGES = 320000
D = 128

NC = 2
NS = 16
NW = NC * NS
EPW = N_EDGES // NW
CH = 100
NCH = EPW // CH
RPS = N_NODES // NS
ZCH = 125


def _sc_agg_body(src_hbm, dst_hbm, x_hbm, out_hbm,
                 srcv, dstv, rows, zbuf, aggsh, sem):
    cid = lax.axis_index("c")
    sid = lax.axis_index("s")
    wid = cid * NS + sid

    def _zrow(r, carry):
        for c in range(D // 16):
            zbuf[r, pl.ds(c * 16, 16)] = jnp.zeros((16,), jnp.float32)
        return carry
    lax.fori_loop(0, ZCH, _zrow, 0)
    for z in range(RPS // ZCH):
        pltpu.sync_copy(zbuf, aggsh.at[pl.ds(sid * RPS + z * ZCH, ZCH)])
    plsc.subcore_barrier()

    pltpu.sync_copy(src_hbm.at[wid], srcv)
    pltpu.sync_copy(dst_hbm.at[wid], dstv)

    def _chunk(i, carry):
        pltpu.async_copy(x_hbm.at[srcv.at[i]], rows, sem).wait()
        pltpu.sync_copy(rows, aggsh.at[dstv.at[i]], add=True)
        return carry
    lax.fori_loop(0, NCH, _chunk, 0)

    plsc.subcore_barrier()
    pltpu.sync_copy(aggsh.at[pl.ds(sid * RPS, RPS)],
                    out_hbm.at[pl.ds(cid * N_NODES + sid * RPS, RPS)])


_sc_agg = functools.partial(
    pl.kernel,
    out_type=jax.ShapeDtypeStruct((NC * N_NODES, D), jnp.float32),
    mesh=plsc.VectorSubcoreMesh(core_axis_name="c", subcore_axis_name="s"),
    scratch_types=[
        pltpu.VMEM((NCH, CH), jnp.int32),
        pltpu.VMEM((NCH, CH), jnp.int32),
        pltpu.VMEM((CH, D), jnp.float32),
        pltpu.VMEM((ZCH, D), jnp.float32),
        pltpu.VMEM_SHARED((N_NODES, D), jnp.float32),
        pltpu.SemaphoreType.DMA,
    ],
    name="sc_edge_segment_sum",
)(_sc_agg_body)


def _mlp_body(x_ref, p0_ref, p1_ref, w1a_ref, w1b_ref, b1_ref,
              gamma_ref, beta_ref, w2_ref, b2_ref, out_ref):
    agg = p0_ref[...] + p1_ref[...]
    h = jnp.dot(x_ref[...], w1a_ref[...], preferred_element_type=jnp.float32)
    h = h + jnp.dot(agg, w1b_ref[...], preferred_element_type=jnp.float32)
    h = jnp.maximum(h + b1_ref[...], 0.0)
    mean = jnp.mean(h, axis=0, keepdims=True)
    cen = h - mean
    var = jnp.mean(cen * cen, axis=0, keepdims=True)
    hn = cen * (lax.rsqrt(var + 1e-5) * gamma_ref[...]) + beta_ref[...]
    out_ref[...] = (
        jnp.dot(hn, w2_ref[...], preferred_element_type=jnp.float32)
        + b2_ref[...])


def kernel(x, edge_index, W1, b1, gamma, beta, W2, b2):
    src = edge_index[0].reshape(NW, NCH, CH)
    dst = edge_index[1].reshape(NW, NCH, CH)
    partials = _sc_agg(src, dst, x)
    p = partials.reshape(NC, N_NODES, D)
    return pl.pallas_call(
        _mlp_body,
        out_shape=jax.ShapeDtypeStruct((N_NODES, D), jnp.float32),
    )(x, p[0], p[1], W1[:D], W1[D:], b1.reshape(1, D),
      gamma.reshape(1, D), beta.reshape(1, D), W2, b2.reshape(1, D))


# speedup vs baseline: 4.9117x; 4.9117x over previous
import functools

import jax
import jax.numpy as jnp
from jax import lax
from jax.experimental import pallas as pl
from jax.experimental.pallas import tpu as pltpu
from jax.experimental.pallas import tpu_sc as plsc

N_NODES = 10000
N_EDGES = 320000
D = 128

NC = 2
NS = 16
HALF = 5120
TRASH = HALF
AROWS = HALF + 8
CH = 128
NCH = 157
EPS = NCH * CH
EPAD = NS * EPS
NPAIR = NCH // 2
RPS = HALF // NS
ZCH = 80


def _sc_agg_body(src_hbm, dst_hbm, x_hbm, out_hbm,
                 srcv, dstv, rows_a, rows_b, zbuf, aggsh, sem_a, sem_b):
    cid = lax.axis_index("c")
    sid = lax.axis_index("s")
    lo = cid * HALF

    def _zrow(r, carry):
        for c in range(D // 16):
            zbuf[r, pl.ds(c * 16, 16)] = jnp.zeros((16,), jnp.float32)
        return carry
    lax.fori_loop(0, ZCH, _zrow, 0)
    for z in range(RPS // ZCH):
        pltpu.sync_copy(zbuf, aggsh.at[pl.ds(sid * RPS + z * ZCH, ZCH)])

    @pl.when(sid == NS - 1)
    def _zero_trash():
        pltpu.sync_copy(zbuf.at[pl.ds(0, 8)], aggsh.at[pl.ds(HALF, 8)])

    plsc.subcore_barrier()

    pltpu.sync_copy(src_hbm.at[sid], srcv)
    pltpu.sync_copy(dst_hbm.at[sid], dstv)

    def _remap(i):
        for j in range(CH // 16):
            t = dstv[i, pl.ds(j * 16, 16)] - lo
            oob = (t < 0) | (t >= HALF)
            dstv[i, pl.ds(j * 16, 16)] = jnp.where(oob, TRASH, t)

    def _pair(p, carry):
        i = 2 * p
        pltpu.async_copy(x_hbm.at[srcv.at[i + 1]], rows_b, sem_b)
        pltpu.make_async_copy(x_hbm.at[srcv.at[i]], rows_a, sem_a).wait()
        _remap(i)
        pltpu.sync_copy(rows_a, aggsh.at[dstv.at[i]], add=True)
        pltpu.async_copy(x_hbm.at[srcv.at[i + 2]], rows_a, sem_a)
        pltpu.make_async_copy(x_hbm.at[srcv.at[i + 1]], rows_b, sem_b).wait()
        _remap(i + 1)
        pltpu.sync_copy(rows_b, aggsh.at[dstv.at[i + 1]], add=True)
        return carry

    pltpu.async_copy(x_hbm.at[srcv.at[0]], rows_a, sem_a)
    lax.fori_loop(0, NPAIR, _pair, 0)
    pltpu.make_async_copy(x_hbm.at[srcv.at[NCH - 1]], rows_a, sem_a).wait()
    _remap(NCH - 1)
    pltpu.sync_copy(rows_a, aggsh.at[dstv.at[NCH - 1]], add=True)

    plsc.subcore_barrier()
    pltpu.sync_copy(aggsh.at[pl.ds(sid * RPS, RPS)],
                    out_hbm.at[pl.ds(cid * HALF + sid * RPS, RPS)])


_sc_agg = functools.partial(
    pl.kernel,
    out_type=jax.ShapeDtypeStruct((NC * HALF, D), jnp.float32),
    mesh=plsc.VectorSubcoreMesh(core_axis_name="c", subcore_axis_name="s"),
    scratch_types=[
        pltpu.VMEM((NCH, CH), jnp.int32),
        pltpu.VMEM((NCH, CH), jnp.int32),
        pltpu.VMEM((CH, D), jnp.float32),
        pltpu.VMEM((CH, D), jnp.float32),
        pltpu.VMEM((ZCH, D), jnp.float32),
        pltpu.VMEM_SHARED((AROWS, D), jnp.float32),
        pltpu.SemaphoreType.DMA,
        pltpu.SemaphoreType.DMA,
    ],
    name="sc_edge_segment_sum",
)(_sc_agg_body)


def _mlp_body(x_ref, agg_ref, w1a_ref, w1b_ref, b1_ref,
              gamma_ref, beta_ref, w2_ref, b2_ref, out_ref):
    h = jnp.dot(x_ref[...], w1a_ref[...], preferred_element_type=jnp.float32)
    h = h + jnp.dot(agg_ref[:N_NODES], w1b_ref[...],
                    preferred_element_type=jnp.float32)
    h = jnp.maximum(h + b1_ref[...], 0.0)
    mean = jnp.mean(h, axis=0, keepdims=True)
    cen = h - mean
    var = jnp.mean(cen * cen, axis=0, keepdims=True)
    hn = cen * (lax.rsqrt(var + 1e-5) * gamma_ref[...]) + beta_ref[...]
    out_ref[...] = (
        jnp.dot(hn, w2_ref[...], preferred_element_type=jnp.float32)
        + b2_ref[...])


def kernel(x, edge_index, W1, b1, gamma, beta, W2, b2):
    npad = EPAD - N_EDGES
    src = jnp.concatenate(
        [edge_index[0], jnp.zeros((npad,), jnp.int32)]).reshape(NS, NCH, CH)
    dst = jnp.concatenate(
        [edge_index[1], jnp.full((npad,), N_NODES, jnp.int32)]
    ).reshape(NS, NCH, CH)
    agg = _sc_agg(src, dst, x)
    return pl.pallas_call(
        _mlp_body,
        out_shape=jax.ShapeDtypeStruct((N_NODES, D), jnp.float32),
    )(x, agg, W1[:D], W1[D:], b1.reshape(1, D),
      gamma.reshape(1, D), beta.reshape(1, D), W2, b2.reshape(1, D))
